# baseline (speedup 1.0000x reference)
import functools

import jax
import jax.numpy as jnp
from jax import lax
from jax.experimental import pallas as pl
from jax.experimental.pallas import tpu as pltpu
from jax.experimental.pallas import tpu_sc as plsc

N = 10000
E = 320000
D = 128
H = 128
C = 10
NUM_LAYERS = 3
G = 64

NC = 2
NS = 16
NW = NC * NS
K = 80
CHUNKS = 125
NGRP = 5
GRP = CHUNKS // NGRP
NBUF = 4
TAIL = GRP - (GRP // NBUF) * NBUF
NPAD = 10240
ROWS_PER_TILE = NPAD // NS

_SC_MESH = plsc.VectorSubcoreMesh(
    core_axis_name="c", subcore_axis_name="s", num_cores=NC, num_subcores=NS)


@functools.partial(
    pl.kernel,
    out_type=jax.ShapeDtypeStruct((NC, NPAD, H), jnp.float32),
    mesh=_SC_MESH,
    scratch_types=[
        pltpu.VMEM((GRP, K), jnp.int32),
        pltpu.VMEM((GRP, K), jnp.int32),
        [pltpu.VMEM((K, H), jnp.float32) for _ in range(NBUF)],
        pltpu.VMEM_SHARED((NPAD, H), jnp.float32),
        pltpu.SemaphoreType.DMA,
    ],
)
def _sc_segment_sum(h_hbm, src_hbm, dst_hbm, zeros_hbm, out_hbm,
                    src_v, dst_v, rows, acc, gsem):
    cid = lax.axis_index("c")
    sid = lax.axis_index("s")
    wid = sid * NC + cid
    z0 = sid * ROWS_PER_TILE
    pltpu.sync_copy(zeros_hbm.at[pl.ds(z0, ROWS_PER_TILE)],
                    acc.at[pl.ds(z0, ROWS_PER_TILE)])
    plsc.subcore_barrier()

    def group(g, carry):
        pltpu.sync_copy(src_hbm.at[wid].at[g], src_v)
        pltpu.sync_copy(dst_hbm.at[wid].at[g], dst_v)
        for b in range(NBUF - 1):
            pltpu.async_copy(h_hbm.at[src_v.at[b]], rows[b], gsem)

        def body(t, carry2):
            for b in range(NBUF):
                j = NBUF * t + b
                pltpu.make_async_copy(h_hbm.at[src_v.at[j]], rows[b],
                                      gsem).wait()

                @pl.when(j + NBUF - 1 < GRP)
                def _():
                    pltpu.async_copy(
                        h_hbm.at[src_v.at[j + NBUF - 1]],
                        rows[(b + NBUF - 1) % NBUF], gsem)

                pltpu.sync_copy(rows[b], acc.at[dst_v.at[j]], add=True)
            return carry2

        lax.fori_loop(0, GRP // NBUF, body, 0)
        for b in range(TAIL):
            j = (GRP // NBUF) * NBUF + b
            pltpu.make_async_copy(h_hbm.at[src_v.at[j]], rows[j % NBUF],
                                  gsem).wait()
            pltpu.sync_copy(rows[j % NBUF], acc.at[dst_v.at[j]], add=True)
        return carry

    lax.fori_loop(0, NGRP, group, 0)
    plsc.subcore_barrier()
    pltpu.sync_copy(acc.at[pl.ds(z0, ROWS_PER_TILE)],
                    out_hbm.at[cid].at[pl.ds(z0, ROWS_PER_TILE)])


_BLK = 1000
_NBLK = N // _BLK
_DEF = lax.Precision.DEFAULT
_HIGH = lax.Precision.HIGHEST


def _bn_coeffs(stats, gamma, beta):
    mu = stats[0:1] / N
    var = stats[1:2] / N - mu * mu
    a = gamma * lax.rsqrt(var + 1e-5)
    return a, beta - a * mu


def _compute_block(agg0_ref, agg1_ref, h_ref, wrel_ref, wroot_ref, brel_ref,
                   batch_ref, stats_scr, psum_scr, cnt_scr, i):
    agg = agg0_ref[0] + agg1_ref[0]
    h = h_ref[...]
    y = lax.dot_general(agg, wrel_ref[...], (((1,), (0,)), ((), ())),
                        precision=_DEF)
    y += lax.dot_general(h, wroot_ref[...], (((1,), (0,)), ((), ())),
                         precision=_DEF)
    y = jnp.maximum(y + brel_ref[...], 0.0)
    onehot = (batch_ref[...] ==
              lax.broadcasted_iota(jnp.int32, (1, G), 1)).astype(jnp.float32)
    ps = lax.dot_general(onehot, y, (((0,), (0,)), ((), ())), precision=_HIGH)
    cnt = lax.dot_general(onehot, jnp.ones((_BLK, 1), jnp.float32),
                          (((0,), (0,)), ((), ())), precision=_HIGH)
    ssum = jnp.sum(y, axis=0, keepdims=True)
    ssq = jnp.sum(y * y, axis=0, keepdims=True)
    st = jnp.concatenate(
        [ssum, ssq, jnp.zeros((6, H), jnp.float32)], axis=0)

    @pl.when(i == 0)
    def _():
        stats_scr[...] = jnp.zeros_like(stats_scr)
        psum_scr[...] = jnp.zeros_like(psum_scr)
        cnt_scr[...] = jnp.zeros_like(cnt_scr)

    stats_scr[...] += st
    psum_scr[...] += ps
    cnt_scr[...] += cnt
    return y


def _head_out(stats_scr, psum_scr, cnt_scr, gamma_ref, beta_ref,
              fcw_ref, fcb_ref, feat_ref, logit_ref):
    a, c = _bn_coeffs(stats_scr[...], gamma_ref[...], beta_ref[...])
    cnt = cnt_scr[...]
    mean = psum_scr[...] / jnp.maximum(cnt, 1.0)
    feat = jnp.where(cnt > 0.0, a * mean + c, 0.0)
    feat_ref[...] = feat
    logit_ref[...] = lax.dot_general(
        feat, fcw_ref[...], (((1,), (0,)), ((), ())),
        precision=_DEF) + fcb_ref[...]


def _fused_body(agg0_ref, agg1_ref, h_ref, wrel_ref, wroot_ref, brel_ref,
                batch_ref, gamma_ref, beta_ref, fcw_ref, fcb_ref,
                hbn_ref, feat_ref, logit_ref,
                y_scr, stats_scr, psum_scr, cnt_scr):
    i = pl.program_id(0)

    @pl.when(i < _NBLK)
    def _():
        y = _compute_block(agg0_ref, agg1_ref, h_ref, wrel_ref, wroot_ref,
                           brel_ref, batch_ref, stats_scr, psum_scr,
                           cnt_scr, i)
        y_scr[pl.ds(i * _BLK, _BLK), :] = y

    @pl.when(i == _NBLK - 1)
    def _():
        _head_out(stats_scr, psum_scr, cnt_scr, gamma_ref, beta_ref,
                  fcw_ref, fcb_ref, feat_ref, logit_ref)

    @pl.when(i >= _NBLK)
    def _():
        a, c = _bn_coeffs(stats_scr[...], gamma_ref[...], beta_ref[...])
        y = y_scr[pl.ds((i - _NBLK) * _BLK, _BLK), :]
        hbn_ref[...] = a * y + c


def _last_body(agg0_ref, agg1_ref, h_ref, wrel_ref, wroot_ref, brel_ref,
               batch_ref, gamma_ref, beta_ref, fcw_ref, fcb_ref,
               feat_ref, logit_ref,
               stats_scr, psum_scr, cnt_scr):
    i = pl.program_id(0)
    _compute_block(agg0_ref, agg1_ref, h_ref, wrel_ref, wroot_ref,
                   brel_ref, batch_ref, stats_scr, psum_scr, cnt_scr, i)

    @pl.when(i == _NBLK - 1)
    def _():
        _head_out(stats_scr, psum_scr, cnt_scr, gamma_ref, beta_ref,
                  fcw_ref, fcb_ref, feat_ref, logit_ref)


def _layer_in_specs(clamped):
    ix = (lambda i: (0, jnp.minimum(i, _NBLK - 1), 0)) if clamped else \
         (lambda i: (0, i, 0))
    ix1 = (lambda i: (1, jnp.minimum(i, _NBLK - 1), 0)) if clamped else \
          (lambda i: (1, i, 0))
    rx = (lambda i: (jnp.minimum(i, _NBLK - 1), 0)) if clamped else \
         (lambda i: (i, 0))
    return [
        pl.BlockSpec((1, _BLK, H), ix),
        pl.BlockSpec((1, _BLK, H), ix1),
        pl.BlockSpec((_BLK, H), rx),
        pl.BlockSpec((H, H), lambda i: (0, 0)),
        pl.BlockSpec((H, H), lambda i: (0, 0)),
        pl.BlockSpec((1, H), lambda i: (0, 0)),
        pl.BlockSpec((_BLK, 1), rx),
        pl.BlockSpec((1, H), lambda i: (0, 0)),
        pl.BlockSpec((1, H), lambda i: (0, 0)),
        pl.BlockSpec((H, C), lambda i: (0, 0)),
        pl.BlockSpec((1, C), lambda i: (0, 0)),
    ]


_fused_layer = pl.pallas_call(
    _fused_body,
    grid=(2 * _NBLK,),
    in_specs=_layer_in_specs(clamped=True),
    out_specs=[
        pl.BlockSpec((_BLK, H), lambda i: (jnp.maximum(i - _NBLK, 0), 0)),
        pl.BlockSpec((G, H), lambda i: (0, 0)),
        pl.BlockSpec((G, C), lambda i: (0, 0)),
    ],
    out_shape=[
        jax.ShapeDtypeStruct((N, H), jnp.float32),
        jax.ShapeDtypeStruct((G, H), jnp.float32),
        jax.ShapeDtypeStruct((G, C), jnp.float32),
    ],
    scratch_shapes=[
        pltpu.VMEM((N, H), jnp.float32),
        pltpu.VMEM((8, H), jnp.float32),
        pltpu.VMEM((G, H), jnp.float32),
        pltpu.VMEM((G, 1), jnp.float32),
    ],
)

_last_layer = pl.pallas_call(
    _last_body,
    grid=(_NBLK,),
    in_specs=_layer_in_specs(clamped=False),
    out_specs=[
        pl.BlockSpec((G, H), lambda i: (0, 0)),
        pl.BlockSpec((G, C), lambda i: (0, 0)),
    ],
    out_shape=[
        jax.ShapeDtypeStruct((G, H), jnp.float32),
        jax.ShapeDtypeStruct((G, C), jnp.float32),
    ],
    scratch_shapes=[
        pltpu.VMEM((8, H), jnp.float32),
        pltpu.VMEM((G, H), jnp.float32),
        pltpu.VMEM((G, 1), jnp.float32),
    ],
)


def kernel(x_paper, W_rel, b_rel, W_root, bn_gamma, bn_beta, fc_W, fc_b,
           edge_index_paper, batch_paper):
    src = edge_index_paper[0]
    dst = edge_index_paper[1]
    src_p = src.reshape(NW, NGRP, GRP, K)
    dst_p = dst.reshape(NW, NGRP, GRP, K)
    zeros_pad = jnp.zeros((NPAD, H), jnp.float32)
    batch2d = batch_paper.reshape(N, 1)
    gamma = bn_gamma.reshape(1, H)
    beta = bn_beta.reshape(1, H)

    h = x_paper
    feats = []
    outs = []
    for l in range(NUM_LAYERS):
        agg_parts = _sc_segment_sum(h, src_p, dst_p, zeros_pad)
        args = (agg_parts, agg_parts, h, W_rel[l], W_root[l],
                b_rel[l].reshape(1, H), batch2d, gamma, beta,
                fc_W[l], fc_b[l].reshape(1, C))
        if l + 1 < NUM_LAYERS:
            h, feat, logit = _fused_layer(*args)
        else:
            feat, logit = _last_layer(*args)
        feats.append(feat)
        outs.append(logit)
    return (tuple(outs), tuple(feats))


# speedup vs baseline: 1.0206x; 1.0206x over previous
import functools

import jax
import jax.numpy as jnp
from jax import lax
from jax.experimental import pallas as pl
from jax.experimental.pallas import tpu as pltpu
from jax.experimental.pallas import tpu_sc as plsc

N = 10000
E = 320000
D = 128
H = 128
C = 10
NUM_LAYERS = 3
G = 64

NC = 2
NS = 16
NW = NC * NS
K = 80
CHUNKS = 125
NGRP = 5
GRP = CHUNKS // NGRP
NBUF = 4
TAIL = GRP - (GRP // NBUF) * NBUF
NPAD = 10240
ROWS_PER_TILE = NPAD // NS

_SC_MESH = plsc.VectorSubcoreMesh(
    core_axis_name="c", subcore_axis_name="s", num_cores=NC, num_subcores=NS)


@functools.partial(
    pl.kernel,
    out_type=jax.ShapeDtypeStruct((NC, NPAD, H), jnp.float32),
    mesh=_SC_MESH,
    scratch_types=[
        pltpu.VMEM((GRP, K), jnp.int32),
        pltpu.VMEM((GRP, K), jnp.int32),
        [pltpu.VMEM((K, H), jnp.float32) for _ in range(NBUF)],
        pltpu.VMEM_SHARED((NPAD, H), jnp.float32),
        pltpu.SemaphoreType.DMA,
    ],
)
def _sc_segment_sum(h_hbm, src_hbm, dst_hbm, zeros_hbm, out_hbm,
                    src_v, dst_v, rows, acc, gsem):
    cid = lax.axis_index("c")
    sid = lax.axis_index("s")
    wid = sid * NC + cid
    z0 = sid * ROWS_PER_TILE
    pltpu.sync_copy(zeros_hbm.at[pl.ds(z0, ROWS_PER_TILE)],
                    acc.at[pl.ds(z0, ROWS_PER_TILE)])
    plsc.subcore_barrier()

    def group(g, carry):
        pltpu.sync_copy(src_hbm.at[wid].at[g], src_v)
        pltpu.sync_copy(dst_hbm.at[wid].at[g], dst_v)
        for b in range(NBUF - 1):
            pltpu.async_copy(h_hbm.at[src_v.at[b]], rows[b], gsem)

        def body(t, carry2):
            for b in range(NBUF):
                j = NBUF * t + b
                pltpu.make_async_copy(h_hbm.at[src_v.at[j]], rows[b],
                                      gsem).wait()

                @pl.when(j + NBUF - 1 < GRP)
                def _():
                    pltpu.async_copy(
                        h_hbm.at[src_v.at[j + NBUF - 1]],
                        rows[(b + NBUF - 1) % NBUF], gsem)

                pltpu.sync_copy(rows[b], acc.at[dst_v.at[j]], add=True)
            return carry2

        lax.fori_loop(0, GRP // NBUF, body, 0)
        for b in range(TAIL):
            j = (GRP // NBUF) * NBUF + b
            pltpu.make_async_copy(h_hbm.at[src_v.at[j]], rows[j % NBUF],
                                  gsem).wait()
            pltpu.sync_copy(rows[j % NBUF], acc.at[dst_v.at[j]], add=True)
        return carry

    lax.fori_loop(0, NGRP, group, 0)
    plsc.subcore_barrier()
    pltpu.sync_copy(acc.at[pl.ds(z0, ROWS_PER_TILE)],
                    out_hbm.at[cid].at[pl.ds(z0, ROWS_PER_TILE)])


_BLK = 2000
_NBLK = N // _BLK
_DEF = lax.Precision.DEFAULT
_HIGH = lax.Precision.HIGHEST


def _bn_coeffs(stats, gamma, beta):
    mu = stats[0:1] / N
    var = stats[1:2] / N - mu * mu
    a = gamma * lax.rsqrt(var + 1e-5)
    return a, beta - a * mu


def _compute_block(agg0_ref, agg1_ref, h_ref, wrel_ref, wroot_ref, brel_ref,
                   batch_ref, stats_scr, psum_scr, cnt_scr, i):
    agg = agg0_ref[0] + agg1_ref[0]
    h = h_ref[...]
    y = lax.dot_general(agg, wrel_ref[...], (((1,), (0,)), ((), ())),
                        precision=_DEF)
    y += lax.dot_general(h, wroot_ref[...], (((1,), (0,)), ((), ())),
                         precision=_DEF)
    y = jnp.maximum(y + brel_ref[...], 0.0)
    onehot = (batch_ref[...] ==
              lax.broadcasted_iota(jnp.int32, (1, G), 1)).astype(jnp.float32)
    ps = lax.dot_general(onehot, y, (((0,), (0,)), ((), ())), precision=_HIGH)
    cnt = lax.dot_general(onehot, jnp.ones((_BLK, 1), jnp.float32),
                          (((0,), (0,)), ((), ())), precision=_HIGH)
    ssum = jnp.sum(y, axis=0, keepdims=True)
    ssq = jnp.sum(y * y, axis=0, keepdims=True)
    st = jnp.concatenate(
        [ssum, ssq, jnp.zeros((6, H), jnp.float32)], axis=0)

    @pl.when(i == 0)
    def _():
        stats_scr[...] = jnp.zeros_like(stats_scr)
        psum_scr[...] = jnp.zeros_like(psum_scr)
        cnt_scr[...] = jnp.zeros_like(cnt_scr)

    stats_scr[...] += st
    psum_scr[...] += ps
    cnt_scr[...] += cnt
    return y


def _head_out(stats_scr, psum_scr, cnt_scr, gamma_ref, beta_ref,
              fcw_ref, fcb_ref, feat_ref, logit_ref):
    a, c = _bn_coeffs(stats_scr[...], gamma_ref[...], beta_ref[...])
    cnt = cnt_scr[...]
    mean = psum_scr[...] / jnp.maximum(cnt, 1.0)
    feat = jnp.where(cnt > 0.0, a * mean + c, 0.0)
    feat_ref[...] = feat
    logit_ref[...] = lax.dot_general(
        feat, fcw_ref[...], (((1,), (0,)), ((), ())),
        precision=_DEF) + fcb_ref[...]


def _fused_body(agg0_ref, agg1_ref, h_ref, wrel_ref, wroot_ref, brel_ref,
                batch_ref, gamma_ref, beta_ref, fcw_ref, fcb_ref,
                hbn_ref, feat_ref, logit_ref,
                y_scr, stats_scr, psum_scr, cnt_scr):
    i = pl.program_id(0)

    @pl.when(i < _NBLK)
    def _():
        y = _compute_block(agg0_ref, agg1_ref, h_ref, wrel_ref, wroot_ref,
                           brel_ref, batch_ref, stats_scr, psum_scr,
                           cnt_scr, i)
        y_scr[pl.ds(i * _BLK, _BLK), :] = y

    @pl.when(i == _NBLK - 1)
    def _():
        _head_out(stats_scr, psum_scr, cnt_scr, gamma_ref, beta_ref,
                  fcw_ref, fcb_ref, feat_ref, logit_ref)

    @pl.when(i >= _NBLK)
    def _():
        a, c = _bn_coeffs(stats_scr[...], gamma_ref[...], beta_ref[...])
        y = y_scr[pl.ds((i - _NBLK) * _BLK, _BLK), :]
        hbn_ref[...] = a * y + c


def _last_body(agg0_ref, agg1_ref, h_ref, wrel_ref, wroot_ref, brel_ref,
               batch_ref, gamma_ref, beta_ref, fcw_ref, fcb_ref,
               feat_ref, logit_ref,
               stats_scr, psum_scr, cnt_scr):
    i = pl.program_id(0)
    _compute_block(agg0_ref, agg1_ref, h_ref, wrel_ref, wroot_ref,
                   brel_ref, batch_ref, stats_scr, psum_scr, cnt_scr, i)

    @pl.when(i == _NBLK - 1)
    def _():
        _head_out(stats_scr, psum_scr, cnt_scr, gamma_ref, beta_ref,
                  fcw_ref, fcb_ref, feat_ref, logit_ref)


def _layer_in_specs(clamped):
    ix = (lambda i: (0, jnp.minimum(i, _NBLK - 1), 0)) if clamped else \
         (lambda i: (0, i, 0))
    ix1 = (lambda i: (1, jnp.minimum(i, _NBLK - 1), 0)) if clamped else \
          (lambda i: (1, i, 0))
    rx = (lambda i: (jnp.minimum(i, _NBLK - 1), 0)) if clamped else \
         (lambda i: (i, 0))
    return [
        pl.BlockSpec((1, _BLK, H), ix),
        pl.BlockSpec((1, _BLK, H), ix1),
        pl.BlockSpec((_BLK, H), rx),
        pl.BlockSpec((H, H), lambda i: (0, 0)),
        pl.BlockSpec((H, H), lambda i: (0, 0)),
        pl.BlockSpec((1, H), lambda i: (0, 0)),
        pl.BlockSpec((_BLK, 1), rx),
        pl.BlockSpec((1, H), lambda i: (0, 0)),
        pl.BlockSpec((1, H), lambda i: (0, 0)),
        pl.BlockSpec((H, C), lambda i: (0, 0)),
        pl.BlockSpec((1, C), lambda i: (0, 0)),
    ]


_fused_layer = pl.pallas_call(
    _fused_body,
    grid=(2 * _NBLK,),
    in_specs=_layer_in_specs(clamped=True),
    out_specs=[
        pl.BlockSpec((_BLK, H), lambda i: (jnp.maximum(i - _NBLK, 0), 0)),
        pl.BlockSpec((G, H), lambda i: (0, 0)),
        pl.BlockSpec((G, C), lambda i: (0, 0)),
    ],
    out_shape=[
        jax.ShapeDtypeStruct((N, H), jnp.float32),
        jax.ShapeDtypeStruct((G, H), jnp.float32),
        jax.ShapeDtypeStruct((G, C), jnp.float32),
    ],
    scratch_shapes=[
        pltpu.VMEM((N, H), jnp.float32),
        pltpu.VMEM((8, H), jnp.float32),
        pltpu.VMEM((G, H), jnp.float32),
        pltpu.VMEM((G, 1), jnp.float32),
    ],
)

_last_layer = pl.pallas_call(
    _last_body,
    grid=(_NBLK,),
    in_specs=_layer_in_specs(clamped=False),
    out_specs=[
        pl.BlockSpec((G, H), lambda i: (0, 0)),
        pl.BlockSpec((G, C), lambda i: (0, 0)),
    ],
    out_shape=[
        jax.ShapeDtypeStruct((G, H), jnp.float32),
        jax.ShapeDtypeStruct((G, C), jnp.float32),
    ],
    scratch_shapes=[
        pltpu.VMEM((8, H), jnp.float32),
        pltpu.VMEM((G, H), jnp.float32),
        pltpu.VMEM((G, 1), jnp.float32),
    ],
)


def kernel(x_paper, W_rel, b_rel, W_root, bn_gamma, bn_beta, fc_W, fc_b,
           edge_index_paper, batch_paper):
    src = edge_index_paper[0]
    dst = edge_index_paper[1]
    src_p = src.reshape(NW, NGRP, GRP, K)
    dst_p = dst.reshape(NW, NGRP, GRP, K)
    zeros_pad = jnp.zeros((NPAD, H), jnp.float32)
    batch2d = batch_paper.reshape(N, 1)
    gamma = bn_gamma.reshape(1, H)
    beta = bn_beta.reshape(1, H)

    h = x_paper
    feats = []
    outs = []
    for l in range(NUM_LAYERS):
        agg_parts = _sc_segment_sum(h, src_p, dst_p, zeros_pad)
        args = (agg_parts, agg_parts, h, W_rel[l], W_root[l],
                b_rel[l].reshape(1, H), batch2d, gamma, beta,
                fc_W[l], fc_b[l].reshape(1, C))
        if l + 1 < NUM_LAYERS:
            h, feat, logit = _fused_layer(*args)
        else:
            feat, logit = _last_layer(*args)
        feats.append(feat)
        outs.append(logit)
    return (tuple(outs), tuple(feats))


# speedup vs baseline: 1.0473x; 1.0262x over previous
import functools

import jax
import jax.numpy as jnp
from jax import lax
from jax.experimental import pallas as pl
from jax.experimental.pallas import tpu as pltpu
from jax.experimental.pallas import tpu_sc as plsc

N = 10000
E = 320000
D = 128
H = 128
C = 10
NUM_LAYERS = 3
G = 64

NC = 2
NS = 16
NW = NC * NS
K = 80
CHUNKS = 125
NGRP = 5
GRP = CHUNKS // NGRP
NBUF = 4
TAIL = GRP - (GRP // NBUF) * NBUF
NPAD = 10240
ROWS_PER_TILE = NPAD // NS

_SC_MESH = plsc.VectorSubcoreMesh(
    core_axis_name="c", subcore_axis_name="s", num_cores=NC, num_subcores=NS)


@functools.partial(
    pl.kernel,
    out_type=jax.ShapeDtypeStruct((NC, NPAD, H), jnp.float32),
    mesh=_SC_MESH,
    scratch_types=[
        pltpu.VMEM((GRP, K), jnp.int32),
        pltpu.VMEM((GRP, K), jnp.int32),
        [pltpu.VMEM((K, H), jnp.float32) for _ in range(NBUF)],
        pltpu.VMEM_SHARED((NPAD, H), jnp.float32),
        pltpu.SemaphoreType.DMA,
        pltpu.SemaphoreType.DMA,
    ],
)
def _sc_segment_sum(h_hbm, src_hbm, dst_hbm, zeros_hbm, out_hbm,
                    src_v, dst_v, rows, acc, gsem, isem):
    cid = lax.axis_index("c")
    sid = lax.axis_index("s")
    wid = sid * NC + cid
    z0 = sid * ROWS_PER_TILE
    pltpu.sync_copy(zeros_hbm.at[pl.ds(z0, ROWS_PER_TILE)],
                    acc.at[pl.ds(z0, ROWS_PER_TILE)])
    plsc.subcore_barrier()

    def group(g, carry):
        csrc = pltpu.async_copy(src_hbm.at[wid].at[g], src_v, isem)
        cdst = pltpu.async_copy(dst_hbm.at[wid].at[g], dst_v, isem)
        csrc.wait()
        for b in range(NBUF - 1):
            pltpu.async_copy(h_hbm.at[src_v.at[b]], rows[b], gsem)
        cdst.wait()

        def body(t, carry2):
            for b in range(NBUF):
                j = NBUF * t + b
                pltpu.make_async_copy(h_hbm.at[src_v.at[j]], rows[b],
                                      gsem).wait()

                @pl.when(j + NBUF - 1 < GRP)
                def _():
                    pltpu.async_copy(
                        h_hbm.at[src_v.at[j + NBUF - 1]],
                        rows[(b + NBUF - 1) % NBUF], gsem)

                pltpu.sync_copy(rows[b], acc.at[dst_v.at[j]], add=True)
            return carry2

        lax.fori_loop(0, GRP // NBUF, body, 0)
        for b in range(TAIL):
            j = (GRP // NBUF) * NBUF + b
            pltpu.make_async_copy(h_hbm.at[src_v.at[j]], rows[j % NBUF],
                                  gsem).wait()
            pltpu.sync_copy(rows[j % NBUF], acc.at[dst_v.at[j]], add=True)
        return carry

    lax.fori_loop(0, NGRP, group, 0)
    plsc.subcore_barrier()
    pltpu.sync_copy(acc.at[pl.ds(z0, ROWS_PER_TILE)],
                    out_hbm.at[cid].at[pl.ds(z0, ROWS_PER_TILE)])


_BLK = 2000
_NBLK = N // _BLK
_DEF = lax.Precision.DEFAULT
_HIGH = lax.Precision.HIGHEST


def _bn_coeffs(stats, gamma, beta):
    mu = stats[0:1] / N
    var = stats[1:2] / N - mu * mu
    a = gamma * lax.rsqrt(var + 1e-5)
    return a, beta - a * mu


def _compute_block(agg0_ref, agg1_ref, h_ref, wrel_ref, wroot_ref, brel_ref,
                   batch_ref, stats_scr, psum_scr, cnt_scr, i):
    agg = agg0_ref[0] + agg1_ref[0]
    h = h_ref[...]
    y = lax.dot_general(agg, wrel_ref[...], (((1,), (0,)), ((), ())),
                        precision=_DEF)
    y += lax.dot_general(h, wroot_ref[...], (((1,), (0,)), ((), ())),
                         precision=_DEF)
    y = jnp.maximum(y + brel_ref[...], 0.0)
    onehot = (batch_ref[...] ==
              lax.broadcasted_iota(jnp.int32, (1, G), 1)).astype(jnp.float32)
    ps = lax.dot_general(onehot, y, (((0,), (0,)), ((), ())), precision=_HIGH)
    cnt = lax.dot_general(onehot, jnp.ones((_BLK, 1), jnp.float32),
                          (((0,), (0,)), ((), ())), precision=_HIGH)
    ssum = jnp.sum(y, axis=0, keepdims=True)
    ssq = jnp.sum(y * y, axis=0, keepdims=True)
    st = jnp.concatenate(
        [ssum, ssq, jnp.zeros((6, H), jnp.float32)], axis=0)

    @pl.when(i == 0)
    def _():
        stats_scr[...] = jnp.zeros_like(stats_scr)
        psum_scr[...] = jnp.zeros_like(psum_scr)
        cnt_scr[...] = jnp.zeros_like(cnt_scr)

    stats_scr[...] += st
    psum_scr[...] += ps
    cnt_scr[...] += cnt
    return y


def _head_out(stats_scr, psum_scr, cnt_scr, gamma_ref, beta_ref,
              fcw_ref, fcb_ref, feat_ref, logit_ref):
    a, c = _bn_coeffs(stats_scr[...], gamma_ref[...], beta_ref[...])
    cnt = cnt_scr[...]
    mean = psum_scr[...] / jnp.maximum(cnt, 1.0)
    feat = jnp.where(cnt > 0.0, a * mean + c, 0.0)
    feat_ref[...] = feat
    logit_ref[...] = lax.dot_general(
        feat, fcw_ref[...], (((1,), (0,)), ((), ())),
        precision=_DEF) + fcb_ref[...]


def _fused_body(agg0_ref, agg1_ref, h_ref, wrel_ref, wroot_ref, brel_ref,
                batch_ref, gamma_ref, beta_ref, fcw_ref, fcb_ref,
                hbn_ref, feat_ref, logit_ref,
                y_scr, stats_scr, psum_scr, cnt_scr):
    i = pl.program_id(0)

    @pl.when(i < _NBLK)
    def _():
        y = _compute_block(agg0_ref, agg1_ref, h_ref, wrel_ref, wroot_ref,
                           brel_ref, batch_ref, stats_scr, psum_scr,
                           cnt_scr, i)
        y_scr[pl.ds(i * _BLK, _BLK), :] = y

    @pl.when(i == _NBLK - 1)
    def _():
        _head_out(stats_scr, psum_scr, cnt_scr, gamma_ref, beta_ref,
                  fcw_ref, fcb_ref, feat_ref, logit_ref)

    @pl.when(i >= _NBLK)
    def _():
        a, c = _bn_coeffs(stats_scr[...], gamma_ref[...], beta_ref[...])
        y = y_scr[pl.ds((i - _NBLK) * _BLK, _BLK), :]
        hbn_ref[...] = a * y + c


def _last_body(agg0_ref, agg1_ref, h_ref, wrel_ref, wroot_ref, brel_ref,
               batch_ref, gamma_ref, beta_ref, fcw_ref, fcb_ref,
               feat_ref, logit_ref,
               stats_scr, psum_scr, cnt_scr):
    i = pl.program_id(0)
    _compute_block(agg0_ref, agg1_ref, h_ref, wrel_ref, wroot_ref,
                   brel_ref, batch_ref, stats_scr, psum_scr, cnt_scr, i)

    @pl.when(i == _NBLK - 1)
    def _():
        _head_out(stats_scr, psum_scr, cnt_scr, gamma_ref, beta_ref,
                  fcw_ref, fcb_ref, feat_ref, logit_ref)


def _layer_in_specs(clamped):
    ix = (lambda i: (0, jnp.minimum(i, _NBLK - 1), 0)) if clamped else \
         (lambda i: (0, i, 0))
    ix1 = (lambda i: (1, jnp.minimum(i, _NBLK - 1), 0)) if clamped else \
          (lambda i: (1, i, 0))
    rx = (lambda i: (jnp.minimum(i, _NBLK - 1), 0)) if clamped else \
         (lambda i: (i, 0))
    return [
        pl.BlockSpec((1, _BLK, H), ix),
        pl.BlockSpec((1, _BLK, H), ix1),
        pl.BlockSpec((_BLK, H), rx),
        pl.BlockSpec((H, H), lambda i: (0, 0)),
        pl.BlockSpec((H, H), lambda i: (0, 0)),
        pl.BlockSpec((1, H), lambda i: (0, 0)),
        pl.BlockSpec((_BLK, 1), rx),
        pl.BlockSpec((1, H), lambda i: (0, 0)),
        pl.BlockSpec((1, H), lambda i: (0, 0)),
        pl.BlockSpec((H, C), lambda i: (0, 0)),
        pl.BlockSpec((1, C), lambda i: (0, 0)),
    ]


_fused_layer = pl.pallas_call(
    _fused_body,
    grid=(2 * _NBLK,),
    in_specs=_layer_in_specs(clamped=True),
    out_specs=[
        pl.BlockSpec((_BLK, H), lambda i: (jnp.maximum(i - _NBLK, 0), 0)),
        pl.BlockSpec((G, H), lambda i: (0, 0)),
        pl.BlockSpec((G, C), lambda i: (0, 0)),
    ],
    out_shape=[
        jax.ShapeDtypeStruct((N, H), jnp.float32),
        jax.ShapeDtypeStruct((G, H), jnp.float32),
        jax.ShapeDtypeStruct((G, C), jnp.float32),
    ],
    scratch_shapes=[
        pltpu.VMEM((N, H), jnp.float32),
        pltpu.VMEM((8, H), jnp.float32),
        pltpu.VMEM((G, H), jnp.float32),
        pltpu.VMEM((G, 1), jnp.float32),
    ],
)

_last_layer = pl.pallas_call(
    _last_body,
    grid=(_NBLK,),
    in_specs=_layer_in_specs(clamped=False),
    out_specs=[
        pl.BlockSpec((G, H), lambda i: (0, 0)),
        pl.BlockSpec((G, C), lambda i: (0, 0)),
    ],
    out_shape=[
        jax.ShapeDtypeStruct((G, H), jnp.float32),
        jax.ShapeDtypeStruct((G, C), jnp.float32),
    ],
    scratch_shapes=[
        pltpu.VMEM((8, H), jnp.float32),
        pltpu.VMEM((G, H), jnp.float32),
        pltpu.VMEM((G, 1), jnp.float32),
    ],
)


def kernel(x_paper, W_rel, b_rel, W_root, bn_gamma, bn_beta, fc_W, fc_b,
           edge_index_paper, batch_paper):
    src = edge_index_paper[0]
    dst = edge_index_paper[1]
    src_p = src.reshape(NW, NGRP, GRP, K)
    dst_p = dst.reshape(NW, NGRP, GRP, K)
    zeros_pad = jnp.zeros((NPAD, H), jnp.float32)
    batch2d = batch_paper.reshape(N, 1)
    gamma = bn_gamma.reshape(1, H)
    beta = bn_beta.reshape(1, H)

    h = x_paper
    feats = []
    outs = []
    for l in range(NUM_LAYERS):
        agg_parts = _sc_segment_sum(h, src_p, dst_p, zeros_pad)
        args = (agg_parts, agg_parts, h, W_rel[l], W_root[l],
                b_rel[l].reshape(1, H), batch2d, gamma, beta,
                fc_W[l], fc_b[l].reshape(1, C))
        if l + 1 < NUM_LAYERS:
            h, feat, logit = _fused_layer(*args)
        else:
            feat, logit = _last_layer(*args)
        feats.append(feat)
        outs.append(logit)
    return (tuple(outs), tuple(feats))


# speedup vs baseline: 1.0629x; 1.0149x over previous
import functools

import jax
import jax.numpy as jnp
from jax import lax
from jax.experimental import pallas as pl
from jax.experimental.pallas import tpu as pltpu
from jax.experimental.pallas import tpu_sc as plsc

N = 10000
E = 320000
D = 128
H = 128
C = 10
NUM_LAYERS = 3
G = 64

NC = 2
NS = 16
NW = NC * NS
K = 80
CHUNKS = 125
NGRP = 5
GRP = CHUNKS // NGRP
NBUF = 4
TAIL = GRP - (GRP // NBUF) * NBUF
NPAD = 10240
ROWS_PER_TILE = NPAD // NS

_SC_MESH = plsc.VectorSubcoreMesh(
    core_axis_name="c", subcore_axis_name="s", num_cores=NC, num_subcores=NS)


@functools.partial(
    pl.kernel,
    out_type=jax.ShapeDtypeStruct((NC, NPAD, H), jnp.float32),
    mesh=_SC_MESH,
    scratch_types=[
        pltpu.VMEM((GRP, K), jnp.int32),
        pltpu.VMEM((GRP, K), jnp.int32),
        [pltpu.VMEM((K, H), jnp.float32) for _ in range(NBUF)],
        pltpu.VMEM_SHARED((NPAD, H), jnp.float32),
        pltpu.SemaphoreType.DMA,
        pltpu.SemaphoreType.DMA,
    ],
)
def _sc_segment_sum(h_hbm, src_hbm, dst_hbm, out_hbm,
                    src_v, dst_v, rows, acc, gsem, isem):
    cid = lax.axis_index("c")
    sid = lax.axis_index("s")
    wid = sid * NC + cid
    z0 = sid * ROWS_PER_TILE

    def zbody(r, carry):
        rows[0][r // 8, pl.ds((r % 8) * 16, 16)] = jnp.zeros((16,),
                                                             jnp.float32)
        return carry

    lax.fori_loop(0, K * 8, zbody, 0)
    for q in range(ROWS_PER_TILE // K):
        pltpu.sync_copy(rows[0], acc.at[pl.ds(z0 + q * K, K)])
    plsc.subcore_barrier()

    def group(g, carry):
        csrc = pltpu.async_copy(src_hbm.at[wid].at[g], src_v, isem)
        cdst = pltpu.async_copy(dst_hbm.at[wid].at[g], dst_v, isem)
        csrc.wait()
        for b in range(NBUF - 1):
            pltpu.async_copy(h_hbm.at[src_v.at[b]], rows[b], gsem)
        cdst.wait()

        def body(t, carry2):
            for b in range(NBUF):
                j = NBUF * t + b
                pltpu.make_async_copy(h_hbm.at[src_v.at[j]], rows[b],
                                      gsem).wait()

                @pl.when(j + NBUF - 1 < GRP)
                def _():
                    pltpu.async_copy(
                        h_hbm.at[src_v.at[j + NBUF - 1]],
                        rows[(b + NBUF - 1) % NBUF], gsem)

                pltpu.sync_copy(rows[b], acc.at[dst_v.at[j]], add=True)
            return carry2

        lax.fori_loop(0, GRP // NBUF, body, 0)
        for b in range(TAIL):
            j = (GRP // NBUF) * NBUF + b
            pltpu.make_async_copy(h_hbm.at[src_v.at[j]], rows[j % NBUF],
                                  gsem).wait()
            pltpu.sync_copy(rows[j % NBUF], acc.at[dst_v.at[j]], add=True)
        return carry

    lax.fori_loop(0, NGRP, group, 0)
    plsc.subcore_barrier()
    pltpu.sync_copy(acc.at[pl.ds(z0, ROWS_PER_TILE)],
                    out_hbm.at[cid].at[pl.ds(z0, ROWS_PER_TILE)])


_BLK = 2000
_NBLK = N // _BLK
_DEF = lax.Precision.DEFAULT
_HIGH = lax.Precision.HIGHEST


def _bn_coeffs(stats, gamma, beta):
    mu = stats[0:1] / N
    var = stats[1:2] / N - mu * mu
    a = gamma * lax.rsqrt(var + 1e-5)
    return a, beta - a * mu


def _compute_block(agg0_ref, agg1_ref, h_ref, wrel_ref, wroot_ref, brel_ref,
                   batch_ref, stats_scr, psum_scr, cnt_scr, i):
    agg = agg0_ref[0] + agg1_ref[0]
    h = h_ref[...]
    y = lax.dot_general(agg, wrel_ref[...], (((1,), (0,)), ((), ())),
                        precision=_DEF)
    y += lax.dot_general(h, wroot_ref[...], (((1,), (0,)), ((), ())),
                         precision=_DEF)
    y = jnp.maximum(y + brel_ref[...], 0.0)
    onehot = (batch_ref[...] ==
              lax.broadcasted_iota(jnp.int32, (1, G), 1)).astype(jnp.float32)
    ps = lax.dot_general(onehot, y, (((0,), (0,)), ((), ())), precision=_HIGH)
    cnt = lax.dot_general(onehot, jnp.ones((_BLK, 1), jnp.float32),
                          (((0,), (0,)), ((), ())), precision=_HIGH)
    ssum = jnp.sum(y, axis=0, keepdims=True)
    ssq = jnp.sum(y * y, axis=0, keepdims=True)
    st = jnp.concatenate(
        [ssum, ssq, jnp.zeros((6, H), jnp.float32)], axis=0)

    @pl.when(i == 0)
    def _():
        stats_scr[...] = jnp.zeros_like(stats_scr)
        psum_scr[...] = jnp.zeros_like(psum_scr)
        cnt_scr[...] = jnp.zeros_like(cnt_scr)

    stats_scr[...] += st
    psum_scr[...] += ps
    cnt_scr[...] += cnt
    return y


def _head_out(stats_scr, psum_scr, cnt_scr, gamma_ref, beta_ref,
              fcw_ref, fcb_ref, feat_ref, logit_ref):
    a, c = _bn_coeffs(stats_scr[...], gamma_ref[...], beta_ref[...])
    cnt = cnt_scr[...]
    mean = psum_scr[...] / jnp.maximum(cnt, 1.0)
    feat = jnp.where(cnt > 0.0, a * mean + c, 0.0)
    feat_ref[...] = feat
    logit_ref[...] = lax.dot_general(
        feat, fcw_ref[...], (((1,), (0,)), ((), ())),
        precision=_DEF) + fcb_ref[...]


def _fused_body(agg0_ref, agg1_ref, h_ref, wrel_ref, wroot_ref, brel_ref,
                batch_ref, gamma_ref, beta_ref, fcw_ref, fcb_ref,
                hbn_ref, feat_ref, logit_ref,
                y_scr, stats_scr, psum_scr, cnt_scr):
    i = pl.program_id(0)

    @pl.when(i < _NBLK)
    def _():
        y = _compute_block(agg0_ref, agg1_ref, h_ref, wrel_ref, wroot_ref,
                           brel_ref, batch_ref, stats_scr, psum_scr,
                           cnt_scr, i)
        y_scr[pl.ds(i * _BLK, _BLK), :] = y

    @pl.when(i == _NBLK - 1)
    def _():
        _head_out(stats_scr, psum_scr, cnt_scr, gamma_ref, beta_ref,
                  fcw_ref, fcb_ref, feat_ref, logit_ref)

    @pl.when(i >= _NBLK)
    def _():
        a, c = _bn_coeffs(stats_scr[...], gamma_ref[...], beta_ref[...])
        y = y_scr[pl.ds((i - _NBLK) * _BLK, _BLK), :]
        hbn_ref[...] = a * y + c


def _last_body(agg0_ref, agg1_ref, h_ref, wrel_ref, wroot_ref, brel_ref,
               batch_ref, gamma_ref, beta_ref, fcw_ref, fcb_ref,
               feat_ref, logit_ref,
               stats_scr, psum_scr, cnt_scr):
    i = pl.program_id(0)
    _compute_block(agg0_ref, agg1_ref, h_ref, wrel_ref, wroot_ref,
                   brel_ref, batch_ref, stats_scr, psum_scr, cnt_scr, i)

    @pl.when(i == _NBLK - 1)
    def _():
        _head_out(stats_scr, psum_scr, cnt_scr, gamma_ref, beta_ref,
                  fcw_ref, fcb_ref, feat_ref, logit_ref)


def _layer_in_specs(clamped):
    ix = (lambda i: (0, jnp.minimum(i, _NBLK - 1), 0)) if clamped else \
         (lambda i: (0, i, 0))
    ix1 = (lambda i: (1, jnp.minimum(i, _NBLK - 1), 0)) if clamped else \
          (lambda i: (1, i, 0))
    rx = (lambda i: (jnp.minimum(i, _NBLK - 1), 0)) if clamped else \
         (lambda i: (i, 0))
    return [
        pl.BlockSpec((1, _BLK, H), ix),
        pl.BlockSpec((1, _BLK, H), ix1),
        pl.BlockSpec((_BLK, H), rx),
        pl.BlockSpec((H, H), lambda i: (0, 0)),
        pl.BlockSpec((H, H), lambda i: (0, 0)),
        pl.BlockSpec((1, H), lambda i: (0, 0)),
        pl.BlockSpec((_BLK, 1), rx),
        pl.BlockSpec((1, H), lambda i: (0, 0)),
        pl.BlockSpec((1, H), lambda i: (0, 0)),
        pl.BlockSpec((H, C), lambda i: (0, 0)),
        pl.BlockSpec((1, C), lambda i: (0, 0)),
    ]


_fused_layer = pl.pallas_call(
    _fused_body,
    grid=(2 * _NBLK,),
    in_specs=_layer_in_specs(clamped=True),
    out_specs=[
        pl.BlockSpec((_BLK, H), lambda i: (jnp.maximum(i - _NBLK, 0), 0)),
        pl.BlockSpec((G, H), lambda i: (0, 0)),
        pl.BlockSpec((G, C), lambda i: (0, 0)),
    ],
    out_shape=[
        jax.ShapeDtypeStruct((N, H), jnp.float32),
        jax.ShapeDtypeStruct((G, H), jnp.float32),
        jax.ShapeDtypeStruct((G, C), jnp.float32),
    ],
    scratch_shapes=[
        pltpu.VMEM((N, H), jnp.float32),
        pltpu.VMEM((8, H), jnp.float32),
        pltpu.VMEM((G, H), jnp.float32),
        pltpu.VMEM((G, 1), jnp.float32),
    ],
)

_last_layer = pl.pallas_call(
    _last_body,
    grid=(_NBLK,),
    in_specs=_layer_in_specs(clamped=False),
    out_specs=[
        pl.BlockSpec((G, H), lambda i: (0, 0)),
        pl.BlockSpec((G, C), lambda i: (0, 0)),
    ],
    out_shape=[
        jax.ShapeDtypeStruct((G, H), jnp.float32),
        jax.ShapeDtypeStruct((G, C), jnp.float32),
    ],
    scratch_shapes=[
        pltpu.VMEM((8, H), jnp.float32),
        pltpu.VMEM((G, H), jnp.float32),
        pltpu.VMEM((G, 1), jnp.float32),
    ],
)


def kernel(x_paper, W_rel, b_rel, W_root, bn_gamma, bn_beta, fc_W, fc_b,
           edge_index_paper, batch_paper):
    src = edge_index_paper[0]
    dst = edge_index_paper[1]
    src_p = src.reshape(NW, NGRP, GRP, K)
    dst_p = dst.reshape(NW, NGRP, GRP, K)
    batch2d = batch_paper.reshape(N, 1)
    gamma = bn_gamma.reshape(1, H)
    beta = bn_beta.reshape(1, H)

    h = x_paper
    feats = []
    outs = []
    for l in range(NUM_LAYERS):
        agg_parts = _sc_segment_sum(h, src_p, dst_p)
        args = (agg_parts, agg_parts, h, W_rel[l], W_root[l],
                b_rel[l].reshape(1, H), batch2d, gamma, beta,
                fc_W[l], fc_b[l].reshape(1, C))
        if l + 1 < NUM_LAYERS:
            h, feat, logit = _fused_layer(*args)
        else:
            feat, logit = _last_layer(*args)
        feats.append(feat)
        outs.append(logit)
    return (tuple(outs), tuple(feats))
